# Initial kernel scaffold; baseline (speedup 1.0000x reference)
import jax
import jax.numpy as jnp
from jax.experimental import pallas as pl


def kernel(node_features, init_adj, W):
    raise NotImplementedError("write your pallas kernel here")


# speedup vs baseline: 3.2474x; 3.2474x over previous
import jax
import jax.numpy as jnp
from jax.experimental import pallas as pl

_N = 4096
_D = 128
_P = 2
_K = _P * _D
_SKIP = 0.8
_TINY = 1e-12
_BM = 256


def _feat_kernel(nf_ref, w_ref, y_ref, yt_ref):
    nf = nf_ref[...]
    scale = 1.0 / jnp.sqrt(jnp.float32(_P))
    cols = []
    for p in range(_P):
        w = w_ref[p, :][None, :]
        ctx = nf * w
        nrm = jnp.sqrt(jnp.sum(ctx * ctx, axis=1, keepdims=True))
        cols.append(ctx / jnp.maximum(nrm, _TINY) * scale)
    y = jnp.concatenate(cols, axis=1)
    y_ref[...] = y
    yt_ref[...] = y.T


def _adj_kernel(yt_ref, y_ref, init_ref, raw_ref, adj_ref):
    yrow = y_ref[...]
    yt = yt_ref[...]
    att = jnp.dot(yrow, yt, preferred_element_type=jnp.float32)
    raw = jnp.maximum(att, 0.0)
    row_sum = jnp.sum(raw, axis=1, keepdims=True)
    inv = 1.0 / jnp.maximum(row_sum, _TINY)
    raw_ref[...] = raw
    adj_ref[...] = _SKIP * init_ref[...] + (1.0 - _SKIP) * (raw * inv)


def kernel(node_features, init_adj, W):
    y, yt = pl.pallas_call(
        _feat_kernel,
        out_shape=(
            jax.ShapeDtypeStruct((_N, _K), jnp.float32),
            jax.ShapeDtypeStruct((_K, _N), jnp.float32),
        ),
    )(node_features, W)

    grid = (_N // _BM,)
    raw, adj = pl.pallas_call(
        _adj_kernel,
        grid=grid,
        in_specs=[
            pl.BlockSpec((_K, _N), lambda i: (0, 0)),
            pl.BlockSpec((_BM, _K), lambda i: (i, 0)),
            pl.BlockSpec((_BM, _N), lambda i: (i, 0)),
        ],
        out_specs=(
            pl.BlockSpec((_BM, _N), lambda i: (i, 0)),
            pl.BlockSpec((_BM, _N), lambda i: (i, 0)),
        ),
        out_shape=(
            jax.ShapeDtypeStruct((_N, _N), jnp.float32),
            jax.ShapeDtypeStruct((_N, _N), jnp.float32),
        ),
    )(yt, y, init_adj)
    return (raw, adj)


# speedup vs baseline: 3.3660x; 1.0365x over previous
import jax
import jax.numpy as jnp
from jax.experimental import pallas as pl
from jax.experimental.pallas import tpu as pltpu

_N = 4096
_D = 128
_P = 2
_K = _P * _D
_SKIP = 0.8
_TINY = 1e-12
_BM = 256


def _feat_kernel(nf_ref, w_ref, y_ref, yt_ref):
    nf = nf_ref[...]
    scale = 1.0 / jnp.sqrt(jnp.float32(_P))
    cols = []
    for p in range(_P):
        w = w_ref[p, :][None, :]
        ctx = nf * w
        nrm = jnp.sqrt(jnp.sum(ctx * ctx, axis=1, keepdims=True))
        cols.append(ctx / jnp.maximum(nrm, _TINY) * scale)
    y = jnp.concatenate(cols, axis=1).astype(jnp.bfloat16)
    y_ref[...] = y
    yt_ref[...] = y.T


def _adj_kernel(yt_ref, y_ref, init_ref, raw_ref, adj_ref):
    yrow = y_ref[...]
    yt = yt_ref[...]
    att = jnp.dot(yrow, yt, preferred_element_type=jnp.float32)
    raw = jnp.maximum(att, 0.0)
    row_sum = jnp.sum(raw, axis=1, keepdims=True)
    inv = 1.0 / jnp.maximum(row_sum, _TINY)
    raw_ref[...] = raw
    adj_ref[...] = _SKIP * init_ref[...] + (1.0 - _SKIP) * (raw * inv)


def kernel(node_features, init_adj, W):
    y, yt = pl.pallas_call(
        _feat_kernel,
        out_shape=(
            jax.ShapeDtypeStruct((_N, _K), jnp.bfloat16),
            jax.ShapeDtypeStruct((_K, _N), jnp.bfloat16),
        ),
    )(node_features, W)

    grid = (_N // _BM,)
    raw, adj = pl.pallas_call(
        _adj_kernel,
        grid=grid,
        in_specs=[
            pl.BlockSpec((_K, _N), lambda i: (0, 0)),
            pl.BlockSpec((_BM, _K), lambda i: (i, 0)),
            pl.BlockSpec((_BM, _N), lambda i: (i, 0)),
        ],
        out_specs=(
            pl.BlockSpec((_BM, _N), lambda i: (i, 0)),
            pl.BlockSpec((_BM, _N), lambda i: (i, 0)),
        ),
        out_shape=(
            jax.ShapeDtypeStruct((_N, _N), jnp.float32),
            jax.ShapeDtypeStruct((_N, _N), jnp.float32),
        ),
        compiler_params=pltpu.CompilerParams(
            dimension_semantics=("parallel",),
        ),
    )(yt, y, init_adj)
    return (raw, adj)


# speedup vs baseline: 3.6221x; 1.0761x over previous
import jax
import jax.numpy as jnp
from jax.experimental import pallas as pl
from jax.experimental.pallas import tpu as pltpu

_N = 4096
_D = 128
_P = 2
_K = _P * _D
_SKIP = 0.8
_TINY = 1e-12
_BM = 256


def _graph_kernel(nf_ref, w_ref, init_ref, raw_ref, adj_ref, y_ref, yt_ref):
    i = pl.program_id(0)

    @pl.when(i == 0)
    def _compute_features():
        nf = nf_ref[...]
        scale = 1.0 / jnp.sqrt(jnp.float32(_P))
        cols = []
        for p in range(_P):
            w = w_ref[p, :][None, :]
            ctx = nf * w
            nrm = jnp.sqrt(jnp.sum(ctx * ctx, axis=1, keepdims=True))
            cols.append(ctx / jnp.maximum(nrm, _TINY) * scale)
        y = jnp.concatenate(cols, axis=1).astype(jnp.bfloat16)
        y_ref[...] = y
        yt_ref[...] = y.T

    yrow = y_ref[pl.ds(i * _BM, _BM), :]
    att = jnp.dot(yrow, yt_ref[...], preferred_element_type=jnp.float32)
    raw = jnp.maximum(att, 0.0)
    row_sum = jnp.sum(raw, axis=1, keepdims=True)
    inv = 1.0 / jnp.maximum(row_sum, _TINY)
    raw_ref[...] = raw
    adj_ref[...] = _SKIP * init_ref[...] + (1.0 - _SKIP) * (raw * inv)


def kernel(node_features, init_adj, W):
    grid = (_N // _BM,)
    raw, adj = pl.pallas_call(
        _graph_kernel,
        grid=grid,
        in_specs=[
            pl.BlockSpec((_N, _D), lambda i: (0, 0)),
            pl.BlockSpec((_P, _D), lambda i: (0, 0)),
            pl.BlockSpec((_BM, _N), lambda i: (i, 0)),
        ],
        out_specs=(
            pl.BlockSpec((_BM, _N), lambda i: (i, 0)),
            pl.BlockSpec((_BM, _N), lambda i: (i, 0)),
        ),
        out_shape=(
            jax.ShapeDtypeStruct((_N, _N), jnp.float32),
            jax.ShapeDtypeStruct((_N, _N), jnp.float32),
        ),
        scratch_shapes=[
            pltpu.VMEM((_N, _K), jnp.bfloat16),
            pltpu.VMEM((_K, _N), jnp.bfloat16),
        ],
    )(node_features, W, init_adj)
    return (raw, adj)
